# baseline (speedup 1.0000x reference)
import functools

import jax
import jax.numpy as jnp
from jax import lax
from jax.experimental import pallas as pl
from jax.experimental.pallas import tpu as pltpu
from jax.experimental.pallas import tpu_sc as plsc

_LANES = 16
_SQRT2 = 1.4142135381698608
_LN2 = 0.6931471805599453


_LOG_CO = (1.4720650105997446e-06, 0.999847697496241, -0.49737321615801344,
           0.3157473167582296, -0.19035433673354185, 0.08269123711181733,
           -0.017414077524385167)


def _ln1p_abs(ad):
    y = ad + 1.0
    bits = lax.bitcast_convert_type(y, jnp.int32)
    e_i = lax.shift_right_logical(bits, 23) - 127
    t = lax.bitcast_convert_type(
        (bits & 0x7FFFFF) | 0x3F800000, jnp.float32) - 1.0
    p = jnp.float32(_LOG_CO[-1])
    for c in _LOG_CO[-2::-1]:
        p = p * t + jnp.float32(c)
    return e_i.astype(jnp.float32) * _LN2 + p


def _make_sc_kernel(N, K):
    KG = 4
    NS = 8
    ROWS = K // KG
    W = 896
    n_phys = -(-N // 128) * 128
    stripe = -(-n_phys // (NS * 128)) * 128
    nblk = stripe // W
    assert stripe % W == 0
    last_len = n_phys - (NS - 1) * stripe
    n_full_last = last_len // W
    tail_w = last_len - n_full_last * W
    assert tail_w % 128 == 0 and tail_w >= 0

    mesh = plsc.VectorSubcoreMesh(core_axis_name="c", subcore_axis_name="s")
    nc = mesh.num_cores

    @functools.partial(
        pl.kernel,
        out_type=jax.ShapeDtypeStruct((3, KG, ROWS, N), jnp.float32),
        mesh=mesh,
        compiler_params=pltpu.CompilerParams(needs_layout_passes=False),
        scratch_types=[
            pltpu.VMEM((n_phys,), jnp.int32),
            pltpu.VMEM((2, ROWS, W), jnp.int32),
            pltpu.VMEM((2, ROWS, W), jnp.float32),
            pltpu.VMEM((2, ROWS, W), jnp.float32),
            pltpu.VMEM((2, ROWS, W), jnp.float32),
            pltpu.SemaphoreType.DMA,
            pltpu.SemaphoreType.DMA,
        ],
    )
    def sc_kernel(c_hbm, e_hbm, out_hbm, c_v, e_v, o0_v, o1_v, o2_v,
                  e_sem, o_sem):
        wid = lax.axis_index("s") * nc + lax.axis_index("c")
        kg = wid % KG
        s = wid // KG
        s_lo = s * stripe
        nblk_t = jnp.where(s == NS - 1, n_full_last, nblk)
        lane = lax.iota(jnp.int32, _LANES)

        def blk_off(b):
            return pl.multiple_of(s_lo + b * W, 128)

        def e_copy(b, ph):
            return pltpu.make_async_copy(
                e_hbm.at[kg, :, pl.ds(blk_off(b), W)], e_v.at[ph], e_sem)

        def o_copies(b, ph):
            n0 = blk_off(b)
            return [
                pltpu.make_async_copy(
                    o_v.at[ph], out_hbm.at[c, kg, :, pl.ds(n0, W)], o_sem)
                for c, o_v in ((0, o0_v), (1, o1_v), (2, o2_v))
            ]

        e_copy(0, 0).start()
        pltpu.sync_copy(c_hbm.at[0], c_v.at[pl.ds(0, N)])

        def compute_block(n0, e_p, o0_p, o1_p, o2_p, nv, clamp):
            @plsc.parallel_loop(0, nv * _LANES, step=_LANES)
            def do_col(col):
                ci = c_v[pl.ds(n0 + col, _LANES)]
                iv = n0 + col + lane
                for r in range(ROWS):
                    jv = e_p[r, pl.ds(col, _LANES)]
                    if clamp:
                        jv = jnp.minimum(jnp.maximum(jv, 0), N - 1)
                    cj = plsc.load_gather(c_v, [jv])
                    d = (jv - iv).astype(jnp.float32)
                    is_if = jnp.where(ci != cj, 1.0, 0.0).astype(jnp.float32)
                    intra = 1.0 - is_if
                    lg = _ln1p_abs(jnp.abs(d))
                    o0_p[r, pl.ds(col, _LANES)] = is_if
                    o1_p[r, pl.ds(col, _LANES)] = intra * lg
                    o2_p[r, pl.ds(col, _LANES)] = intra * jnp.sign(d)

        last_common = n_full_last
        for b in range(nblk):
            ph = b % 2
            gate = None if b < last_common else (s < NS - 1)

            def step(b=b, ph=ph):
                e_copy(b, ph).wait()
                nxt = b + 1
                if nxt < nblk:
                    if nxt < last_common:
                        e_copy(nxt, 1 - ph).start()
                    else:
                        @pl.when(s < NS - 1)
                        def _():
                            e_copy(nxt, 1 - ph).start()
                if b >= 2:
                    for cpy in o_copies(b - 2, ph):
                        cpy.wait()
                compute_block(blk_off(b), e_v.at[ph], o0_v.at[ph],
                              o1_v.at[ph], o2_v.at[ph], W // _LANES,
                              clamp=False)
                for cpy in o_copies(b, ph):
                    cpy.start()

            if gate is None:
                step()
            else:
                pl.when(gate)(step)

        if nblk >= 2:
            @pl.when(s == NS - 1)
            def _():
                for cpy in o_copies(last_common - 2, (last_common - 2) % 2):
                    cpy.wait()
        for b in range(max(0, nblk - 2), nblk):
            if b >= last_common:
                @pl.when(s < NS - 1)
                def _(b=b):
                    for cpy in o_copies(b, b % 2):
                        cpy.wait()
            else:
                for cpy in o_copies(b, b % 2):
                    cpy.wait()

        if tail_w:
            @pl.when(s == NS - 1)
            def _():
                n0t = pl.multiple_of(s_lo + n_full_last * W, 128)
                pltpu.sync_copy(e_hbm.at[kg, :, pl.ds(n0t, tail_w)],
                                e_v.at[0, :, pl.ds(0, tail_w)])
                compute_block(n0t, e_v.at[0], o0_v.at[0], o1_v.at[0],
                              o2_v.at[0], tail_w // _LANES, clamp=True)
                for c, o_v in ((0, o0_v), (1, o1_v), (2, o2_v)):
                    pltpu.sync_copy(
                        o_v.at[0, :, pl.ds(0, tail_w)],
                        out_hbm.at[c, kg, :, pl.ds(n0t, tail_w)])

    return sc_kernel


def kernel(X, edge_idx, C):
    B, N, K = edge_idx.shape
    assert B == 1 and K % 4 == 0 and (K // 4) % 8 == 0 and N % _LANES == 0
    e_t = jnp.transpose(edge_idx, (0, 2, 1)).reshape(4, K // 4, N)
    out = _make_sc_kernel(N, K)(C, e_t)
    return jnp.transpose(out.reshape(1, 3, K, N), (0, 3, 2, 1))


# speedup vs baseline: 1.0385x; 1.0385x over previous
import functools

import jax
import jax.numpy as jnp
from jax import lax
from jax.experimental import pallas as pl
from jax.experimental.pallas import tpu as pltpu
from jax.experimental.pallas import tpu_sc as plsc

_LANES = 16
_SQRT2 = 1.4142135381698608
_LN2 = 0.6931471805599453


_LOG_K = 8.262958294867817e-08
_LOG_CO = (-88.0296224853685, 3.6134096106749784e-08,
           -6.6285457283409845e-15, 3.7043373817841695e-22,
           -1.1199919217028697e-29)


def _ln1p_abs(ad_i):
    y = ad_i.astype(jnp.float32) + 1.0
    bits = lax.bitcast_convert_type(y, jnp.int32)
    fb = bits.astype(jnp.float32)
    u = (bits & 0x7FFFFF).astype(jnp.float32)
    p = jnp.float32(_LOG_CO[-1])
    for c in _LOG_CO[-2::-1]:
        p = p * u + jnp.float32(c)
    return fb * jnp.float32(_LOG_K) + p


def _make_sc_kernel(N, K):
    KG = 4
    NS = 8
    ROWS = K // KG
    W = 896
    n_phys = -(-N // 128) * 128
    stripe = -(-n_phys // (NS * 128)) * 128
    nblk = stripe // W
    assert stripe % W == 0
    last_len = n_phys - (NS - 1) * stripe
    n_full_last = last_len // W
    tail_w = last_len - n_full_last * W
    assert tail_w % 128 == 0 and tail_w >= 0

    mesh = plsc.VectorSubcoreMesh(core_axis_name="c", subcore_axis_name="s")
    nc = mesh.num_cores

    @functools.partial(
        pl.kernel,
        out_type=jax.ShapeDtypeStruct((3, KG, ROWS, N), jnp.float32),
        mesh=mesh,
        compiler_params=pltpu.CompilerParams(needs_layout_passes=False),
        scratch_types=[
            pltpu.VMEM((n_phys,), jnp.int32),
            pltpu.VMEM((2, ROWS, W), jnp.int32),
            pltpu.VMEM((2, ROWS, W), jnp.float32),
            pltpu.VMEM((2, ROWS, W), jnp.float32),
            pltpu.VMEM((2, ROWS, W), jnp.float32),
            pltpu.SemaphoreType.DMA,
            pltpu.SemaphoreType.DMA,
        ],
    )
    def sc_kernel(c_hbm, e_hbm, out_hbm, c_v, e_v, o0_v, o1_v, o2_v,
                  e_sem, o_sem):
        wid = lax.axis_index("s") * nc + lax.axis_index("c")
        kg = wid % KG
        s = wid // KG
        s_lo = s * stripe
        nblk_t = jnp.where(s == NS - 1, n_full_last, nblk)
        lane = lax.iota(jnp.int32, _LANES)

        def blk_off(b):
            return pl.multiple_of(s_lo + b * W, 128)

        def e_copy(b, ph):
            return pltpu.make_async_copy(
                e_hbm.at[kg, :, pl.ds(blk_off(b), W)], e_v.at[ph], e_sem)

        def o_copies(b, ph):
            n0 = blk_off(b)
            return [
                pltpu.make_async_copy(
                    o_v.at[ph], out_hbm.at[c, kg, :, pl.ds(n0, W)], o_sem)
                for c, o_v in ((0, o0_v), (1, o1_v), (2, o2_v))
            ]

        e_copy(0, 0).start()
        pltpu.sync_copy(c_hbm.at[0], c_v.at[pl.ds(0, N)])

        def compute_block(n0, e_p, o0_p, o1_p, o2_p, nv, clamp):
            @plsc.parallel_loop(0, nv * _LANES, step=_LANES)
            def do_col(col):
                ci = c_v[pl.ds(n0 + col, _LANES)]
                iv = n0 + col + lane
                for r in range(ROWS):
                    jv = e_p[r, pl.ds(col, _LANES)]
                    if clamp:
                        jv = jnp.minimum(jnp.maximum(jv, 0), N - 1)
                    cj = plsc.load_gather(c_v, [jv])
                    d = jv - iv
                    same = ci == cj
                    is_if = jnp.where(same, 0.0, 1.0).astype(jnp.float32)
                    intra = jnp.where(same, 1.0, 0.0).astype(jnp.float32)
                    lg = _ln1p_abs(jnp.abs(d))
                    sgn = jnp.minimum(jnp.maximum(d, -1), 1).astype(jnp.float32)
                    o0_p[r, pl.ds(col, _LANES)] = is_if
                    o1_p[r, pl.ds(col, _LANES)] = intra * lg
                    o2_p[r, pl.ds(col, _LANES)] = intra * sgn

        last_common = n_full_last
        for b in range(nblk):
            ph = b % 2
            gate = None if b < last_common else (s < NS - 1)

            def step(b=b, ph=ph):
                e_copy(b, ph).wait()
                nxt = b + 1
                if nxt < nblk:
                    if nxt < last_common:
                        e_copy(nxt, 1 - ph).start()
                    else:
                        @pl.when(s < NS - 1)
                        def _():
                            e_copy(nxt, 1 - ph).start()
                if b >= 2:
                    for cpy in o_copies(b - 2, ph):
                        cpy.wait()
                compute_block(blk_off(b), e_v.at[ph], o0_v.at[ph],
                              o1_v.at[ph], o2_v.at[ph], W // _LANES,
                              clamp=False)
                for cpy in o_copies(b, ph):
                    cpy.start()

            if gate is None:
                step()
            else:
                pl.when(gate)(step)

        if nblk >= 2:
            @pl.when(s == NS - 1)
            def _():
                for cpy in o_copies(last_common - 2, (last_common - 2) % 2):
                    cpy.wait()
        for b in range(max(0, nblk - 2), nblk):
            if b >= last_common:
                @pl.when(s < NS - 1)
                def _(b=b):
                    for cpy in o_copies(b, b % 2):
                        cpy.wait()
            else:
                for cpy in o_copies(b, b % 2):
                    cpy.wait()

        if tail_w:
            @pl.when(s == NS - 1)
            def _():
                n0t = pl.multiple_of(s_lo + n_full_last * W, 128)
                pltpu.sync_copy(e_hbm.at[kg, :, pl.ds(n0t, tail_w)],
                                e_v.at[0, :, pl.ds(0, tail_w)])
                compute_block(n0t, e_v.at[0], o0_v.at[0], o1_v.at[0],
                              o2_v.at[0], tail_w // _LANES, clamp=True)
                for c, o_v in ((0, o0_v), (1, o1_v), (2, o2_v)):
                    pltpu.sync_copy(
                        o_v.at[0, :, pl.ds(0, tail_w)],
                        out_hbm.at[c, kg, :, pl.ds(n0t, tail_w)])

    return sc_kernel


def kernel(X, edge_idx, C):
    B, N, K = edge_idx.shape
    assert B == 1 and K % 4 == 0 and (K // 4) % 8 == 0 and N % _LANES == 0
    e_t = jnp.transpose(edge_idx, (0, 2, 1)).reshape(4, K // 4, N)
    out = _make_sc_kernel(N, K)(C, e_t)
    return jnp.transpose(out.reshape(1, 3, K, N), (0, 3, 2, 1))


# speedup vs baseline: 1.0792x; 1.0392x over previous
import functools

import jax
import jax.numpy as jnp
from jax import lax
from jax.experimental import pallas as pl
from jax.experimental.pallas import tpu as pltpu
from jax.experimental.pallas import tpu_sc as plsc

_LANES = 16
_SQRT2 = 1.4142135381698608
_LN2 = 0.6931471805599453


_LOG_K = 8.262958294867817e-08
_LOG_CO = (-88.0296224853685, 3.6134096106749784e-08,
           -6.6285457283409845e-15, 3.7043373817841695e-22,
           -1.1199919217028697e-29)


def _ln1p_abs(ad_i):
    y = ad_i.astype(jnp.float32) + 1.0
    bits = lax.bitcast_convert_type(y, jnp.int32)
    fb = bits.astype(jnp.float32)
    u = (bits & 0x7FFFFF).astype(jnp.float32)
    p = jnp.float32(_LOG_CO[-1])
    for c in _LOG_CO[-2::-1]:
        p = p * u + jnp.float32(c)
    return fb * jnp.float32(_LOG_K) + p


def _make_sc_kernel(N, K):
    KG = 4
    NS = 8
    ROWS = K // KG
    W = 896
    n_phys = -(-N // 128) * 128
    stripe = -(-n_phys // (NS * 128)) * 128
    nblk = stripe // W
    assert stripe % W == 0
    last_len = n_phys - (NS - 1) * stripe
    n_full_last = last_len // W
    tail_w = last_len - n_full_last * W
    assert tail_w % 128 == 0 and tail_w >= 0

    mesh = plsc.VectorSubcoreMesh(core_axis_name="c", subcore_axis_name="s")
    nc = mesh.num_cores

    @functools.partial(
        pl.kernel,
        out_type=jax.ShapeDtypeStruct((3, KG, ROWS, N), jnp.float32),
        mesh=mesh,
        compiler_params=pltpu.CompilerParams(needs_layout_passes=False),
        scratch_types=[
            pltpu.VMEM((n_phys,), jnp.int32),
            pltpu.VMEM((2, ROWS, W), jnp.int32),
            pltpu.VMEM((2, ROWS, W), jnp.float32),
            pltpu.VMEM((2, ROWS, W), jnp.float32),
            pltpu.VMEM((2, ROWS, W), jnp.float32),
            pltpu.SemaphoreType.DMA,
            pltpu.SemaphoreType.DMA,
        ],
    )
    def sc_kernel(c_hbm, e_hbm, out_hbm, c_v, e_v, o0_v, o1_v, o2_v,
                  e_sem, o_sem):
        wid = lax.axis_index("s") * nc + lax.axis_index("c")
        kg = wid % KG
        s = wid // KG
        s_lo = s * stripe
        nblk_t = jnp.where(s == NS - 1, n_full_last, nblk)
        lane = lax.iota(jnp.int32, _LANES)

        def blk_off(b):
            return pl.multiple_of(s_lo + b * W, 128)

        def e_copy(b, ph):
            return pltpu.make_async_copy(
                e_hbm.at[kg, :, pl.ds(blk_off(b), W)], e_v.at[ph], e_sem)

        def o_copies(b, ph):
            n0 = blk_off(b)
            return [
                pltpu.make_async_copy(
                    o_v.at[ph], out_hbm.at[c, kg, :, pl.ds(n0, W)], o_sem)
                for c, o_v in ((0, o0_v), (1, o1_v), (2, o2_v))
            ]

        e_copy(0, 0).start()
        pltpu.sync_copy(c_hbm.at[0], c_v.at[pl.ds(0, N)])

        def compute_block(n0, e_p, o0_p, o1_p, o2_p, nv, clamp):
            @plsc.parallel_loop(0, nv * _LANES, step=_LANES)
            def do_col(col):
                ci = c_v[pl.ds(n0 + col, _LANES)]
                iv = n0 + col + lane
                for r in range(ROWS):
                    jv = e_p[r, pl.ds(col, _LANES)]
                    if clamp:
                        jv = jnp.minimum(jnp.maximum(jv, 0), N - 1)
                    cj = plsc.load_gather(c_v, [jv])
                    d = jv - iv
                    same = ci == cj
                    lg = _ln1p_abs(jnp.abs(d))
                    sgn = jnp.minimum(jnp.maximum(d, -1), 1).astype(jnp.float32)
                    o0_p[r, pl.ds(col, _LANES)] = jnp.where(same, 0.0, 1.0)
                    o1_p[r, pl.ds(col, _LANES)] = jnp.where(same, lg, 0.0)
                    o2_p[r, pl.ds(col, _LANES)] = jnp.where(same, sgn, 0.0)

        last_common = n_full_last
        for b in range(nblk):
            ph = b % 2
            gate = None if b < last_common else (s < NS - 1)

            def step(b=b, ph=ph):
                e_copy(b, ph).wait()
                nxt = b + 1
                if nxt < nblk:
                    if nxt < last_common:
                        e_copy(nxt, 1 - ph).start()
                    else:
                        @pl.when(s < NS - 1)
                        def _():
                            e_copy(nxt, 1 - ph).start()
                if b >= 2:
                    for cpy in o_copies(b - 2, ph):
                        cpy.wait()
                compute_block(blk_off(b), e_v.at[ph], o0_v.at[ph],
                              o1_v.at[ph], o2_v.at[ph], W // _LANES,
                              clamp=False)
                for cpy in o_copies(b, ph):
                    cpy.start()

            if gate is None:
                step()
            else:
                pl.when(gate)(step)

        if nblk >= 2:
            @pl.when(s == NS - 1)
            def _():
                for cpy in o_copies(last_common - 2, (last_common - 2) % 2):
                    cpy.wait()
        for b in range(max(0, nblk - 2), nblk):
            if b >= last_common:
                @pl.when(s < NS - 1)
                def _(b=b):
                    for cpy in o_copies(b, b % 2):
                        cpy.wait()
            else:
                for cpy in o_copies(b, b % 2):
                    cpy.wait()

        if tail_w:
            @pl.when(s == NS - 1)
            def _():
                n0t = pl.multiple_of(s_lo + n_full_last * W, 128)
                pltpu.sync_copy(e_hbm.at[kg, :, pl.ds(n0t, tail_w)],
                                e_v.at[0, :, pl.ds(0, tail_w)])
                compute_block(n0t, e_v.at[0], o0_v.at[0], o1_v.at[0],
                              o2_v.at[0], tail_w // _LANES, clamp=True)
                for c, o_v in ((0, o0_v), (1, o1_v), (2, o2_v)):
                    pltpu.sync_copy(
                        o_v.at[0, :, pl.ds(0, tail_w)],
                        out_hbm.at[c, kg, :, pl.ds(n0t, tail_w)])

    return sc_kernel


def kernel(X, edge_idx, C):
    B, N, K = edge_idx.shape
    assert B == 1 and K % 4 == 0 and (K // 4) % 8 == 0 and N % _LANES == 0
    e_t = jnp.transpose(edge_idx, (0, 2, 1)).reshape(4, K // 4, N)
    out = _make_sc_kernel(N, K)(C, e_t)
    return jnp.transpose(out.reshape(1, 3, K, N), (0, 3, 2, 1))
